# baseline (speedup 1.0000x reference)
import functools

import jax
import jax.numpy as jnp
from jax import lax
from jax.experimental import pallas as pl
from jax.experimental.pallas import tpu as pltpu
from jax.experimental.pallas import tpu_sc as plsc

_N = 10000
_E = 320000

_NC = 2
_NS = 16
_NW = _NC * _NS
_RB = 160
_NBKT = 65
_NPAD = 64 * _RB
_EPT = _NPAD
_E_PAD = _NW * _EPT
_SB = 1280
_CK = 256
_CW = 2 * _CK
_TSLOT = _EPT // _CK + 1
_MAXC = _NS * _TSLOT


def _bucket_body(src_hbm, dst_hbm, chunks_out, counts_out,
                 st_s, st_d, bs_s, bs_d, fb, cbcnt, fl_smem, nc_smem):
    scid = lax.axis_index("c")
    sid = lax.axis_index("s")
    wid = sid * _NC + scid
    zi = jnp.zeros((16,), jnp.int32)
    iv = lax.iota(jnp.int32, 16)

    for b in range(_NBKT):
        fl_smem[b] = 0
        nc_smem[b] = 0

    def blk(bi, c):
        base = wid * _EPT + bi * _SB
        pltpu.sync_copy(src_hbm.at[pl.ds(base, _SB)], st_s.at[pl.ds(0, _SB)])
        pltpu.sync_copy(dst_hbm.at[pl.ds(base, _SB)], st_d.at[pl.ds(0, _SB)])

        def eb(e, c2):
            s_e = st_s[pl.ds(e, 16)][0]
            d_e = st_d[pl.ds(e, 16)][0]
            b = d_e // _RB
            dl = d_e - b * _RB
            fill = fl_smem[b]
            sub = fill >> 6
            fo = fill & 63
            bs_s[b, sub, pl.ds(fo, 16)] = zi + s_e
            bs_d[b, sub, pl.ds(fo, 16)] = zi + dl

            def flush():
                for j in range(_CK // 16):
                    sl = pl.ds((j % 4) * 16, 16)
                    fb[pl.ds(j * 16, 16)] = bs_s[b, j // 4, sl]
                    fb[pl.ds(_CK + j * 16, 16)] = bs_d[b, j // 4, sl]
                ncb = nc_smem[b]
                pltpu.sync_copy(fb, chunks_out.at[scid, b, sid * _TSLOT + ncb])
                nc_smem[b] = ncb + 1
                return 0

            nf = lax.cond(fill + 1 >= _CK, flush, lambda: fill + 1)
            fl_smem[b] = nf
            return c2

        lax.fori_loop(0, _SB, eb, 0)
        return c

    lax.fori_loop(0, _EPT // _SB, blk, 0)

    for b in range(_NBKT):
        cbcnt[pl.ds(b, 16)] = zi + (nc_smem[b] * _CK + fl_smem[b])
    pltpu.sync_copy(cbcnt, counts_out.at[scid, sid])

    def fin(b, c):
        fill = fl_smem[b]

        def do():
            for j in range(_CK // 16):
                sl = pl.ds((j % 4) * 16, 16)
                m = (iv + j * 16) < fill
                fb[pl.ds(j * 16, 16)] = jnp.where(m, bs_s[b, j // 4, sl], 0)
                fb[pl.ds(_CK + j * 16, 16)] = jnp.where(m, bs_d[b, j // 4, sl], _RB)
            ncb = nc_smem[b]
            pltpu.sync_copy(fb, chunks_out.at[scid, b, sid * _TSLOT + ncb])
            nc_smem[b] = ncb + 1
            return 0

        return lax.cond(fill > 0, do, lambda: 0) * 0 + c

    lax.fori_loop(0, _NBKT, fin, 0)


def _bucketize(src, dst):
    mesh = plsc.VectorSubcoreMesh(core_axis_name="c", subcore_axis_name="s",
                                  num_cores=_NC, num_subcores=_NS)
    i32 = jnp.int32
    kern = pl.kernel(
        _bucket_body,
        out_type=(
            jax.ShapeDtypeStruct((_NC, _NBKT, _MAXC, _CW), i32),
            jax.ShapeDtypeStruct((_NC, _NS, 128), i32),
        ),
        mesh=mesh,
        scratch_types=[
            pltpu.VMEM((_SB + 16,), i32),
            pltpu.VMEM((_SB + 16,), i32),
            pltpu.VMEM((_NBKT, _CK // 64, 80), i32),
            pltpu.VMEM((_NBKT, _CK // 64, 80), i32),
            pltpu.VMEM((_CW,), i32),
            pltpu.VMEM((128,), i32),
            pltpu.SMEM((80,), i32),
            pltpu.SMEM((80,), i32),
        ],
    )
    return kern(src, dst)


def _mom_body(P, xv, chunks, counts, s_out, q_out, mx_out, mn_out, deg_out,
              cntf, cbuf, gidx, dlb, rows, sa, qa, xa, na, dg):
    scid = lax.axis_index("c")
    sid = lax.axis_index("s")
    t = sid * _NC + scid
    pltpu.sync_copy(counts, cntf)
    zf = jnp.zeros((16,), jnp.float32)
    ninf = jnp.full((16,), -jnp.inf, jnp.float32)
    pinf = jnp.full((16,), jnp.inf, jnp.float32)
    onehot = jnp.where(lax.iota(jnp.int32, 16) == 0, 1.0, 0.0)

    def do_bucket(p, b):
        def ms(r, c):
            for j in range(8):
                sl = pl.ds(j * 16, 16)
                sa[r, sl] = zf
                qa[r, sl] = zf
                xa[r, sl] = ninf
                na[r, sl] = pinf
            return c

        lax.fori_loop(0, _RB + 1, ms, 0)

        def msd(i, c):
            dg[pl.ds(i * 16, 16)] = zf
            return c

        lax.fori_loop(0, 16, msd, 0)

        for scsrc in range(_NC):
            def tl_body(tl, c):
                ec = cntf[pl.ds(scsrc * 2048 + tl * 128 + b, 16)][0]
                nch = (ec + _CK - 1) // _CK

                def ch_body(ch, c2):
                    pltpu.sync_copy(chunks.at[scsrc, b, tl * _TSLOT + ch], cbuf)
                    for j in range(_CK // 16):
                        sl = pl.ds(j * 16, 16)
                        gidx[sl] = cbuf[sl] * P + p
                        dlb[sl] = cbuf[pl.ds(_CK + j * 16, 16)]
                    pltpu.sync_copy(xv.at[gidx], rows)
                    m = jnp.where(ch == nch - 1, ec - (nch - 1) * _CK, _CK)

                    def eb(e, c3):
                        dl = dlb[pl.ds(e, 16)][0]
                        for j in range(8):
                            sl = pl.ds(j * 16, 16)
                            v = rows[e, sl]
                            plsc.addupdate(sa.at[dl, sl], v)
                            plsc.addupdate(qa.at[dl, sl], v * v)
                            xa[dl, sl] = jnp.maximum(xa[dl, sl], v)
                            na[dl, sl] = jnp.minimum(na[dl, sl], v)
                        plsc.addupdate(dg.at[pl.ds(dl, 16)], onehot)
                        return c3

                    return lax.fori_loop(0, m, eb, c2)

                return lax.fori_loop(0, nch, ch_body, c)

            lax.fori_loop(0, _NS, tl_body, 0)

        lo = b * _RB
        pltpu.sync_copy(sa.at[pl.ds(0, _RB)], s_out.at[p, pl.ds(lo, _RB)])
        pltpu.sync_copy(qa.at[pl.ds(0, _RB)], q_out.at[p, pl.ds(lo, _RB)])
        pltpu.sync_copy(xa.at[pl.ds(0, _RB)], mx_out.at[p, pl.ds(lo, _RB)])
        pltpu.sync_copy(na.at[pl.ds(0, _RB)], mn_out.at[p, pl.ds(lo, _RB)])

        def wd():
            pltpu.sync_copy(dg.at[pl.ds(0, 256)], deg_out.at[b])
            return 0

        lax.cond(p == 0, wd, lambda: 0)

    def p_body(p, c):
        for k in range(2):
            do_bucket(p, t + 32 * k)
        return c

    lax.fori_loop(0, P, p_body, 0)


def _moments(xv, chunks, counts, P):
    mesh = plsc.VectorSubcoreMesh(core_axis_name="c", subcore_axis_name="s",
                                  num_cores=_NC, num_subcores=_NS)
    f32 = jnp.float32
    i32 = jnp.int32
    kern = pl.kernel(
        functools.partial(_mom_body, P),
        out_type=(
            jax.ShapeDtypeStruct((P, _NPAD, 128), f32),
            jax.ShapeDtypeStruct((P, _NPAD, 128), f32),
            jax.ShapeDtypeStruct((P, _NPAD, 128), f32),
            jax.ShapeDtypeStruct((P, _NPAD, 128), f32),
            jax.ShapeDtypeStruct((_NBKT, 256), f32),
        ),
        mesh=mesh,
        scratch_types=[
            pltpu.VMEM((4096,), i32),
            pltpu.VMEM((_CW,), i32),
            pltpu.VMEM((_CK,), i32),
            pltpu.VMEM((_CK + 16,), i32),
            pltpu.VMEM((_CK, 128), f32),
            pltpu.VMEM((_RB + 1, 128), f32),
            pltpu.VMEM((_RB + 1, 128), f32),
            pltpu.VMEM((_RB + 1, 128), f32),
            pltpu.VMEM((_RB + 1, 128), f32),
            pltpu.VMEM((256,), f32),
        ],
    )
    return kern(xv, chunks, counts)


def _to_full(a, P):
    if P == 1:
        return a[0, :_N]
    return jnp.moveaxis(a[:, :_N, :], 0, 1).reshape(_N, P * 128)


def _post_body(sum_ref, sq_ref, mx_ref, mn_ref, deg_ref, s1_ref, s2_ref,
               w_ref, b_ref, out_ref, *, relu):
    deg_c = jnp.maximum(deg_ref[...], 1.0)
    mean = sum_ref[...] / deg_c
    var = jnp.maximum(sq_ref[...] / deg_c - mean * mean, 0.0)
    std = jnp.sqrt(var + 1e-5)
    mx = mx_ref[...]
    mn = mn_ref[...]
    mx = jnp.where(jnp.isfinite(mx), mx, 0.0)
    mn = jnp.where(jnp.isfinite(mn), mn, 0.0)
    agg = jnp.concatenate([mean, mn, mx, std], axis=-1)
    m0 = jnp.dot(agg, w_ref[0], preferred_element_type=jnp.float32)
    m1 = jnp.dot(agg, w_ref[1], preferred_element_type=jnp.float32)
    m2 = jnp.dot(agg, w_ref[2], preferred_element_type=jnp.float32)
    out = m0 + s1_ref[...] * m1 + s2_ref[...] * m2 + b_ref[...]
    if relu:
        out = jnp.maximum(out, 0.0)
    out_ref[...] = out


def _pna_post(s, sq, mx, mn, deg, s1, s2, W, b, *, relu):
    n, c = s.shape
    out = W.shape[-1]
    w3 = W.reshape(3, 4 * c, out)
    b2 = b.reshape(1, out)
    R = 1000
    grid = (n // R,)
    row = lambda i: (i, 0)
    return pl.pallas_call(
        functools.partial(_post_body, relu=relu),
        grid=grid,
        in_specs=[
            pl.BlockSpec((R, c), row),
            pl.BlockSpec((R, c), row),
            pl.BlockSpec((R, c), row),
            pl.BlockSpec((R, c), row),
            pl.BlockSpec((R, 1), row),
            pl.BlockSpec((R, 1), row),
            pl.BlockSpec((R, 1), row),
            pl.BlockSpec((3, 4 * c, out), lambda i: (0, 0, 0)),
            pl.BlockSpec((1, out), lambda i: (0, 0)),
        ],
        out_specs=pl.BlockSpec((R, out), row),
        out_shape=jax.ShapeDtypeStruct((n, out), jnp.float32),
    )(s, sq, mx, mn, deg, s1, s2, w3, b2)


def kernel(x, edge_index, W1, b1, W2, b2):
    x = x.astype(jnp.float32)
    i32 = jnp.int32
    src = edge_index[0].astype(i32)
    dst = edge_index[1].astype(i32)
    pad = _E_PAD - _E
    srcp = jnp.concatenate([src, jnp.zeros((pad,), i32)])
    dstp = jnp.concatenate([dst, jnp.full((pad,), _NPAD, i32)])

    chunks, counts = _bucketize(srcp, dstp)
    counts = counts.reshape(-1)

    s, q, mx, mn, degr = _moments(x, chunks, counts, 1)
    deg = degr[:64, :_RB].reshape(_NPAD)[:_N]
    log_deg = jnp.log(deg + 1.0)
    delta = jnp.mean(log_deg)
    s1 = (log_deg / delta)[:, None]
    s2 = (delta / jnp.clip(log_deg, 1e-5))[:, None]
    degc = deg[:, None]

    h = _pna_post(_to_full(s, 1), _to_full(q, 1), _to_full(mx, 1),
                  _to_full(mn, 1), degc, s1, s2, W1, b1, relu=True)

    s, q, mx, mn, _ = _moments(h.reshape(_N * 2, 128), chunks, counts, 2)
    return _pna_post(_to_full(s, 2), _to_full(q, 2), _to_full(mx, 2),
                     _to_full(mn, 2), degc, s1, s2, W2, b2, relu=False)


# speedup vs baseline: 2.9176x; 2.9176x over previous
import functools

import jax
import jax.numpy as jnp
from jax import lax
from jax.experimental import pallas as pl
from jax.experimental.pallas import tpu as pltpu
from jax.experimental.pallas import tpu_sc as plsc

_N = 10000
_E = 320000

_NC = 2
_NS = 16
_NW = _NC * _NS
_RB = 160
_NBKT = 65
_NPAD = 64 * _RB
_EPT = _NPAD
_E_PAD = _NW * _EPT
_SB = 1280
_CK = 64
_TSLOT = 161
_MAXC = _NS * _TSLOT


def _bucket_body(src_hbm, dst_hbm, chunks_out, counts_out,
                 st_s, st_d, bs_s, bs_d, fb, cbcnt, fl_smem, nc_smem):
    scid = lax.axis_index("c")
    sid = lax.axis_index("s")
    wid = sid * _NC + scid
    zi = jnp.zeros((16,), jnp.int32)
    iv = lax.iota(jnp.int32, 16)

    for b in range(_NBKT):
        fl_smem[b] = 0
        nc_smem[b] = 0

    def blk(bi, c):
        base = wid * _EPT + bi * _SB
        pltpu.sync_copy(src_hbm.at[pl.ds(base, _SB)], st_s.at[pl.ds(0, _SB)])
        pltpu.sync_copy(dst_hbm.at[pl.ds(base, _SB)], st_d.at[pl.ds(0, _SB)])

        def eb(e, c2):
            s_e = st_s[pl.ds(e, 16)][0]
            d_e = st_d[pl.ds(e, 16)][0]
            b = d_e // _RB
            dl = d_e - b * _RB
            fill = fl_smem[b]
            bs_s[b, pl.ds(fill, 16)] = zi + s_e
            bs_d[b, pl.ds(fill, 16)] = zi + dl

            def flush():
                for j in range(4):
                    sl = pl.ds(j * 16, 16)
                    fb[sl] = bs_s[b, sl]
                    fb[pl.ds(64 + j * 16, 16)] = bs_d[b, sl]
                ncb = nc_smem[b]
                pltpu.sync_copy(fb, chunks_out.at[scid, b, sid * _TSLOT + ncb])
                nc_smem[b] = ncb + 1
                return 0

            nf = lax.cond(fill + 1 >= _CK, flush, lambda: fill + 1)
            fl_smem[b] = nf
            return c2

        lax.fori_loop(0, _SB, eb, 0)
        return c

    lax.fori_loop(0, _EPT // _SB, blk, 0)

    for b in range(_NBKT):
        cbcnt[pl.ds(b, 16)] = zi + (nc_smem[b] * _CK + fl_smem[b])
    pltpu.sync_copy(cbcnt, counts_out.at[scid, sid])

    def fin(b, c):
        fill = fl_smem[b]

        def do():
            for j in range(4):
                sl = pl.ds(j * 16, 16)
                m = (iv + j * 16) < fill
                fb[sl] = jnp.where(m, bs_s[b, sl], 0)
                fb[pl.ds(64 + j * 16, 16)] = jnp.where(m, bs_d[b, sl], _RB)
            ncb = nc_smem[b]
            pltpu.sync_copy(fb, chunks_out.at[scid, b, sid * _TSLOT + ncb])
            nc_smem[b] = ncb + 1
            return 0

        return lax.cond(fill > 0, do, lambda: 0) * 0 + c

    lax.fori_loop(0, _NBKT, fin, 0)


def _bucketize(src, dst):
    mesh = plsc.VectorSubcoreMesh(core_axis_name="c", subcore_axis_name="s",
                                  num_cores=_NC, num_subcores=_NS)
    i32 = jnp.int32
    kern = pl.kernel(
        _bucket_body,
        out_type=(
            jax.ShapeDtypeStruct((_NC, _NBKT, _MAXC, 128), i32),
            jax.ShapeDtypeStruct((_NC, _NS, 128), i32),
        ),
        mesh=mesh,
        scratch_types=[
            pltpu.VMEM((_SB + 16,), i32),
            pltpu.VMEM((_SB + 16,), i32),
            pltpu.VMEM((_NBKT, 80), i32),
            pltpu.VMEM((_NBKT, 80), i32),
            pltpu.VMEM((128,), i32),
            pltpu.VMEM((128,), i32),
            pltpu.SMEM((80,), i32),
            pltpu.SMEM((80,), i32),
        ],
    )
    return kern(src, dst)


def _mom_body(P, xv, chunks, counts, s_out, q_out, mx_out, mn_out, deg_out,
              cntf, cbuf, gidx, dlb, rows, sa, qa, xa, na, dg):
    scid = lax.axis_index("c")
    sid = lax.axis_index("s")
    t = sid * _NC + scid
    pltpu.sync_copy(counts, cntf)
    zf = jnp.zeros((16,), jnp.float32)
    ninf = jnp.full((16,), -jnp.inf, jnp.float32)
    pinf = jnp.full((16,), jnp.inf, jnp.float32)
    onehot = jnp.where(lax.iota(jnp.int32, 16) == 0, 1.0, 0.0)

    def do_bucket(p, b):
        def ms(r, c):
            for j in range(8):
                sl = pl.ds(j * 16, 16)
                sa[r, sl] = zf
                qa[r, sl] = zf
                xa[r, sl] = ninf
                na[r, sl] = pinf
            return c

        lax.fori_loop(0, _RB + 16, ms, 0)

        def msd(i, c):
            dg[pl.ds(i * 16, 16)] = zf
            return c

        lax.fori_loop(0, 17, msd, 0)

        for scsrc in range(_NC):
            def tl_body(tl, c):
                ec = cntf[pl.ds(scsrc * 2048 + tl * 128 + b, 16)][0]
                nch = (ec + _CK - 1) // _CK

                def ch_body(ch, c2):
                    pltpu.sync_copy(chunks.at[scsrc, b, tl * _TSLOT + ch], cbuf)
                    for j in range(4):
                        sl = pl.ds(j * 16, 16)
                        gidx[sl] = cbuf[sl] * P + p
                        dlb[sl] = cbuf[pl.ds(64 + j * 16, 16)]
                    pltpu.sync_copy(xv.at[gidx], rows)
                    m = jnp.where(ch == nch - 1, ec - (nch - 1) * _CK, _CK)

                    def eb(e, c3):
                        dl = dlb[pl.ds(e, 16)][0]
                        for j in range(8):
                            sl = pl.ds(j * 16, 16)
                            v = rows[e, sl]
                            plsc.addupdate(sa.at[dl, sl], v)
                            plsc.addupdate(qa.at[dl, sl], v * v)
                            xa[dl, sl] = jnp.maximum(xa[dl, sl], v)
                            na[dl, sl] = jnp.minimum(na[dl, sl], v)
                        plsc.addupdate(dg.at[pl.ds(dl, 16)], onehot)
                        return c3

                    return lax.fori_loop(0, m, eb, c2)

                return lax.fori_loop(0, nch, ch_body, c)

            lax.fori_loop(0, _NS, tl_body, 0)

        lo = b * _RB
        pltpu.sync_copy(sa.at[pl.ds(0, _RB)], s_out.at[p, pl.ds(lo, _RB)])
        pltpu.sync_copy(qa.at[pl.ds(0, _RB)], q_out.at[p, pl.ds(lo, _RB)])
        pltpu.sync_copy(xa.at[pl.ds(0, _RB)], mx_out.at[p, pl.ds(lo, _RB)])
        pltpu.sync_copy(na.at[pl.ds(0, _RB)], mn_out.at[p, pl.ds(lo, _RB)])

        def wd():
            pltpu.sync_copy(dg.at[pl.ds(0, 256)], deg_out.at[b])
            return 0

        lax.cond(p == 0, wd, lambda: 0)

    def p_body(p, c):
        for k in range(2):
            do_bucket(p, t + 32 * k)
        return c

    lax.fori_loop(0, P, p_body, 0)


def _moments(xv, chunks, counts, P):
    mesh = plsc.VectorSubcoreMesh(core_axis_name="c", subcore_axis_name="s",
                                  num_cores=_NC, num_subcores=_NS)
    f32 = jnp.float32
    i32 = jnp.int32
    kern = pl.kernel(
        functools.partial(_mom_body, P),
        out_type=(
            jax.ShapeDtypeStruct((P, _NPAD, 128), f32),
            jax.ShapeDtypeStruct((P, _NPAD, 128), f32),
            jax.ShapeDtypeStruct((P, _NPAD, 128), f32),
            jax.ShapeDtypeStruct((P, _NPAD, 128), f32),
            jax.ShapeDtypeStruct((_NBKT, 256), f32),
        ),
        mesh=mesh,
        scratch_types=[
            pltpu.VMEM((4096,), i32),
            pltpu.VMEM((128,), i32),
            pltpu.VMEM((64,), i32),
            pltpu.VMEM((80,), i32),
            pltpu.VMEM((_CK, 128), f32),
            pltpu.VMEM((_RB + 16, 128), f32),
            pltpu.VMEM((_RB + 16, 128), f32),
            pltpu.VMEM((_RB + 16, 128), f32),
            pltpu.VMEM((_RB + 16, 128), f32),
            pltpu.VMEM((272,), f32),
        ],
    )
    return kern(xv, chunks, counts)


def _to_full(a, P):
    if P == 1:
        return a[0, :_N]
    return jnp.moveaxis(a[:, :_N, :], 0, 1).reshape(_N, P * 128)


def _post_body(sum_ref, sq_ref, mx_ref, mn_ref, deg_ref, s1_ref, s2_ref,
               w_ref, b_ref, out_ref, *, relu):
    deg_c = jnp.maximum(deg_ref[...], 1.0)
    mean = sum_ref[...] / deg_c
    var = jnp.maximum(sq_ref[...] / deg_c - mean * mean, 0.0)
    std = jnp.sqrt(var + 1e-5)
    mx = mx_ref[...]
    mn = mn_ref[...]
    mx = jnp.where(jnp.isfinite(mx), mx, 0.0)
    mn = jnp.where(jnp.isfinite(mn), mn, 0.0)
    agg = jnp.concatenate([mean, mn, mx, std], axis=-1)
    m0 = jnp.dot(agg, w_ref[0], preferred_element_type=jnp.float32)
    m1 = jnp.dot(agg, w_ref[1], preferred_element_type=jnp.float32)
    m2 = jnp.dot(agg, w_ref[2], preferred_element_type=jnp.float32)
    out = m0 + s1_ref[...] * m1 + s2_ref[...] * m2 + b_ref[...]
    if relu:
        out = jnp.maximum(out, 0.0)
    out_ref[...] = out


def _pna_post(s, sq, mx, mn, deg, s1, s2, W, b, *, relu):
    n, c = s.shape
    out = W.shape[-1]
    w3 = W.reshape(3, 4 * c, out)
    b2 = b.reshape(1, out)
    R = 1000
    grid = (n // R,)
    row = lambda i: (i, 0)
    return pl.pallas_call(
        functools.partial(_post_body, relu=relu),
        grid=grid,
        in_specs=[
            pl.BlockSpec((R, c), row),
            pl.BlockSpec((R, c), row),
            pl.BlockSpec((R, c), row),
            pl.BlockSpec((R, c), row),
            pl.BlockSpec((R, 1), row),
            pl.BlockSpec((R, 1), row),
            pl.BlockSpec((R, 1), row),
            pl.BlockSpec((3, 4 * c, out), lambda i: (0, 0, 0)),
            pl.BlockSpec((1, out), lambda i: (0, 0)),
        ],
        out_specs=pl.BlockSpec((R, out), row),
        out_shape=jax.ShapeDtypeStruct((n, out), jnp.float32),
    )(s, sq, mx, mn, deg, s1, s2, w3, b2)


def kernel(x, edge_index, W1, b1, W2, b2):
    x = x.astype(jnp.float32)
    i32 = jnp.int32
    src = edge_index[0].astype(i32)
    dst = edge_index[1].astype(i32)
    pad = _E_PAD - _E
    srcp = jnp.concatenate([src, jnp.zeros((pad,), i32)])
    dstp = jnp.concatenate([dst, jnp.full((pad,), _NPAD, i32)])

    chunks, counts = _bucketize(srcp, dstp)
    counts = counts.reshape(-1)

    s, q, mx, mn, degr = _moments(x, chunks, counts, 1)
    deg = degr[:64, :_RB].reshape(_NPAD)[:_N]
    log_deg = jnp.log(deg + 1.0)
    delta = jnp.mean(log_deg)
    s1 = (log_deg / delta)[:, None]
    s2 = (delta / jnp.clip(log_deg, 1e-5))[:, None]
    degc = deg[:, None]

    h = _pna_post(_to_full(s, 1), _to_full(q, 1), _to_full(mx, 1),
                  _to_full(mn, 1), degc, s1, s2, W1, b1, relu=True)

    s, q, mx, mn, _ = _moments(h.reshape(_N * 2, 128), chunks, counts, 2)
    return _pna_post(_to_full(s, 2), _to_full(q, 2), _to_full(mx, 2),
                     _to_full(mn, 2), degc, s1, s2, W2, b2, relu=False)


# speedup vs baseline: 2.9602x; 1.0146x over previous
import functools

import jax
import jax.numpy as jnp
from jax import lax
from jax.experimental import pallas as pl
from jax.experimental.pallas import tpu as pltpu
from jax.experimental.pallas import tpu_sc as plsc

_N = 10000
_E = 320000

_NC = 2
_NS = 16
_NW = _NC * _NS
_RB = 160
_NBKT = 65
_NPAD = 64 * _RB
_EPT = _NPAD
_E_PAD = _NW * _EPT
_SB = 1280
_CK = 64
_TSLOT = 161
_MAXC = _NS * _TSLOT


def _bucket_body(src_hbm, dst_hbm, chunks_out, counts_out,
                 st_s, st_d, bs_s, bs_d, fb, cbcnt, fl_smem, nc_smem):
    scid = lax.axis_index("c")
    sid = lax.axis_index("s")
    wid = sid * _NC + scid
    zi = jnp.zeros((16,), jnp.int32)
    iv = lax.iota(jnp.int32, 16)

    for b in range(_NBKT):
        fl_smem[b] = 0
        nc_smem[b] = 0

    def blk(bi, c):
        base = wid * _EPT + bi * _SB
        pltpu.sync_copy(src_hbm.at[pl.ds(base, _SB)], st_s.at[pl.ds(0, _SB)])
        pltpu.sync_copy(dst_hbm.at[pl.ds(base, _SB)], st_d.at[pl.ds(0, _SB)])

        def eb(e, c2):
            s_e = st_s[pl.ds(e, 16)][0]
            d_e = st_d[pl.ds(e, 16)][0]
            b = d_e // _RB
            dl = d_e - b * _RB
            fill = fl_smem[b]
            bs_s[b, pl.ds(fill, 16)] = zi + s_e
            bs_d[b, pl.ds(fill, 16)] = zi + dl

            def flush():
                for j in range(4):
                    sl = pl.ds(j * 16, 16)
                    fb[sl] = bs_s[b, sl]
                    fb[pl.ds(64 + j * 16, 16)] = bs_d[b, sl]
                ncb = nc_smem[b]
                pltpu.sync_copy(fb, chunks_out.at[scid, b, sid * _TSLOT + ncb])
                nc_smem[b] = ncb + 1
                return 0

            nf = lax.cond(fill + 1 >= _CK, flush, lambda: fill + 1)
            fl_smem[b] = nf
            return c2

        lax.fori_loop(0, _SB, eb, 0)
        return c

    lax.fori_loop(0, _EPT // _SB, blk, 0)

    for b in range(_NBKT):
        cbcnt[pl.ds(b, 16)] = zi + (nc_smem[b] * _CK + fl_smem[b])
    pltpu.sync_copy(cbcnt, counts_out.at[scid, sid])

    def fin(b, c):
        fill = fl_smem[b]

        def do():
            for j in range(4):
                sl = pl.ds(j * 16, 16)
                m = (iv + j * 16) < fill
                fb[sl] = jnp.where(m, bs_s[b, sl], 0)
                fb[pl.ds(64 + j * 16, 16)] = jnp.where(m, bs_d[b, sl], _RB)
            ncb = nc_smem[b]
            pltpu.sync_copy(fb, chunks_out.at[scid, b, sid * _TSLOT + ncb])
            nc_smem[b] = ncb + 1
            return 0

        return lax.cond(fill > 0, do, lambda: 0) * 0 + c

    lax.fori_loop(0, _NBKT, fin, 0)


def _bucketize(src, dst):
    mesh = plsc.VectorSubcoreMesh(core_axis_name="c", subcore_axis_name="s",
                                  num_cores=_NC, num_subcores=_NS)
    i32 = jnp.int32
    kern = pl.kernel(
        _bucket_body,
        out_type=(
            jax.ShapeDtypeStruct((_NC, _NBKT, _MAXC, 128), i32),
            jax.ShapeDtypeStruct((_NC, _NS, 128), i32),
        ),
        mesh=mesh,
        scratch_types=[
            pltpu.VMEM((_SB + 16,), i32),
            pltpu.VMEM((_SB + 16,), i32),
            pltpu.VMEM((_NBKT, 80), i32),
            pltpu.VMEM((_NBKT, 80), i32),
            pltpu.VMEM((128,), i32),
            pltpu.VMEM((128,), i32),
            pltpu.SMEM((80,), i32),
            pltpu.SMEM((80,), i32),
        ],
    )
    return kern(src, dst)


def _mom_body(P, xv, chunks, counts, s_out, q_out, mx_out, mn_out, deg_out,
              cntf, wl_a, wl_m, cbufs, gidx, dlb, rows, sa, qa, xa, na, dg,
              semca, semcb, semga, semgb):
    scid = lax.axis_index("c")
    sid = lax.axis_index("s")
    t = sid * _NC + scid
    pltpu.sync_copy(counts, cntf)
    zf = jnp.zeros((16,), jnp.float32)
    zi = jnp.zeros((16,), jnp.int32)
    ninf = jnp.full((16,), -jnp.inf, jnp.float32)
    pinf = jnp.full((16,), jnp.inf, jnp.float32)
    onehot = jnp.where(lax.iota(jnp.int32, 16) == 0, 1.0, 0.0)

    def do_bucket(p, b):
        def ms(r, c):
            for j in range(8):
                sl = pl.ds(j * 16, 16)
                sa[r, sl] = zf
                qa[r, sl] = zf
                xa[r, sl] = ninf
                na[r, sl] = pinf
            return c

        lax.fori_loop(0, _RB + 16, ms, 0)

        def msd(i, c):
            dg[pl.ds(i * 16, 16)] = zf
            return c

        lax.fori_loop(0, 17, msd, 0)

        ntot = 0
        for scsrc in range(_NC):
            def tl_b(tl, n):
                ec = cntf[pl.ds(scsrc * 2048 + tl * 128 + b, 16)][0]
                nch = (ec + _CK - 1) // _CK

                def cb(ch, n2):
                    wl_a[pl.ds(n2, 16)] = zi + (scsrc * 65536 + tl * _TSLOT + ch)
                    wl_m[pl.ds(n2, 16)] = zi + jnp.where(
                        ch == nch - 1, ec - (nch - 1) * _CK, _CK)
                    return n2 + 1

                return lax.fori_loop(0, nch, cb, n)

            ntot = lax.fori_loop(0, _NS, tl_b, ntot)

        def fetch(i, par):
            a = wl_a[pl.ds(i, 16)][0]
            sc = a >> 16
            slot = a & 0xFFFF

            def fa():
                pltpu.async_copy(chunks.at[sc, b, slot], cbufs.at[0], semca)
                return 0

            def fb():
                pltpu.async_copy(chunks.at[sc, b, slot], cbufs.at[1], semcb)
                return 0

            lax.cond(par == 0, fa, fb)

        def wait_cbuf(par):
            def wa():
                pltpu.make_async_copy(chunks.at[0, 0, 0], cbufs.at[0],
                                      semca).wait()
                return 0

            def wb():
                pltpu.make_async_copy(chunks.at[0, 0, 0], cbufs.at[1],
                                      semcb).wait()
                return 0

            lax.cond(par == 0, wa, wb)

        def build_gather(par):
            def ga():
                for j in range(4):
                    sl = pl.ds(j * 16, 16)
                    gidx[0, sl] = cbufs[0, sl] * P + p
                    dlb[0, sl] = cbufs[0, pl.ds(64 + j * 16, 16)]
                pltpu.async_copy(xv.at[gidx.at[0]], rows.at[0], semga)
                return 0

            def gb():
                for j in range(4):
                    sl = pl.ds(j * 16, 16)
                    gidx[1, sl] = cbufs[1, sl] * P + p
                    dlb[1, sl] = cbufs[1, pl.ds(64 + j * 16, 16)]
                pltpu.async_copy(xv.at[gidx.at[1]], rows.at[1], semgb)
                return 0

            lax.cond(par == 0, ga, gb)

        def wait_gather(par):
            def wa():
                pltpu.make_async_copy(xv.at[gidx.at[0]], rows.at[0],
                                      semga).wait()
                return 0

            def wb():
                pltpu.make_async_copy(xv.at[gidx.at[1]], rows.at[1],
                                      semgb).wait()
                return 0

            lax.cond(par == 0, wa, wb)

        def process(i, par):
            m = wl_m[pl.ds(i, 16)][0]

            def eb(e, c3):
                dl = dlb[par, pl.ds(e, 16)][0]
                for j in range(8):
                    sl = pl.ds(j * 16, 16)
                    v = rows[par, e, sl]
                    plsc.addupdate(sa.at[dl, sl], v)
                    plsc.addupdate(qa.at[dl, sl], v * v)
                    xa[dl, sl] = jnp.maximum(xa[dl, sl], v)
                    na[dl, sl] = jnp.minimum(na[dl, sl], v)
                plsc.addupdate(dg.at[pl.ds(dl, 16)], onehot)
                return c3

            lax.fori_loop(0, m, eb, 0)

        def have_work():
            fetch(0, 0)

            def it(i, c):
                par = i & 1
                nxt = 1 - par
                wait_cbuf(par)
                build_gather(par)
                lax.cond(i + 1 < ntot,
                         lambda: (fetch(i + 1, nxt), 0)[1], lambda: 0)
                lax.cond(i > 0,
                         lambda: (wait_gather(nxt), process(i - 1, nxt), 0)[2],
                         lambda: 0)
                return c

            lax.fori_loop(0, ntot, it, 0)
            lastpar = (ntot - 1) & 1
            wait_gather(lastpar)
            process(ntot - 1, lastpar)
            return 0

        lax.cond(ntot > 0, have_work, lambda: 0)

        lo = b * _RB
        pltpu.sync_copy(sa.at[pl.ds(0, _RB)], s_out.at[p, pl.ds(lo, _RB)])
        pltpu.sync_copy(qa.at[pl.ds(0, _RB)], q_out.at[p, pl.ds(lo, _RB)])
        pltpu.sync_copy(xa.at[pl.ds(0, _RB)], mx_out.at[p, pl.ds(lo, _RB)])
        pltpu.sync_copy(na.at[pl.ds(0, _RB)], mn_out.at[p, pl.ds(lo, _RB)])

        def wd():
            pltpu.sync_copy(dg.at[pl.ds(0, 256)], deg_out.at[b])
            return 0

        lax.cond(p == 0, wd, lambda: 0)

    def p_body(p, c):
        for k in range(2):
            do_bucket(p, t + 32 * k)
        return c

    lax.fori_loop(0, P, p_body, 0)


def _moments(xv, chunks, counts, P):
    mesh = plsc.VectorSubcoreMesh(core_axis_name="c", subcore_axis_name="s",
                                  num_cores=_NC, num_subcores=_NS)
    f32 = jnp.float32
    i32 = jnp.int32
    kern = pl.kernel(
        functools.partial(_mom_body, P),
        out_type=(
            jax.ShapeDtypeStruct((P, _NPAD, 128), f32),
            jax.ShapeDtypeStruct((P, _NPAD, 128), f32),
            jax.ShapeDtypeStruct((P, _NPAD, 128), f32),
            jax.ShapeDtypeStruct((P, _NPAD, 128), f32),
            jax.ShapeDtypeStruct((_NBKT, 256), f32),
        ),
        mesh=mesh,
        scratch_types=[
            pltpu.VMEM((4096,), i32),
            pltpu.VMEM((5168,), i32),
            pltpu.VMEM((5168,), i32),
            pltpu.VMEM((2, 128), i32),
            pltpu.VMEM((2, 64), i32),
            pltpu.VMEM((2, 80), i32),
            pltpu.VMEM((2, _CK, 128), f32),
            pltpu.VMEM((_RB + 16, 128), f32),
            pltpu.VMEM((_RB + 16, 128), f32),
            pltpu.VMEM((_RB + 16, 128), f32),
            pltpu.VMEM((_RB + 16, 128), f32),
            pltpu.VMEM((272,), f32),
            pltpu.SemaphoreType.DMA,
            pltpu.SemaphoreType.DMA,
            pltpu.SemaphoreType.DMA,
            pltpu.SemaphoreType.DMA,
        ],
    )
    return kern(xv, chunks, counts)


def _to_full(a, P):
    if P == 1:
        return a[0, :_N]
    return jnp.moveaxis(a[:, :_N, :], 0, 1).reshape(_N, P * 128)


def _post_body(sum_ref, sq_ref, mx_ref, mn_ref, deg_ref, s1_ref, s2_ref,
               w_ref, b_ref, out_ref, *, relu):
    deg_c = jnp.maximum(deg_ref[...], 1.0)
    mean = sum_ref[...] / deg_c
    var = jnp.maximum(sq_ref[...] / deg_c - mean * mean, 0.0)
    std = jnp.sqrt(var + 1e-5)
    mx = mx_ref[...]
    mn = mn_ref[...]
    mx = jnp.where(jnp.isfinite(mx), mx, 0.0)
    mn = jnp.where(jnp.isfinite(mn), mn, 0.0)
    agg = jnp.concatenate([mean, mn, mx, std], axis=-1)
    m0 = jnp.dot(agg, w_ref[0], preferred_element_type=jnp.float32)
    m1 = jnp.dot(agg, w_ref[1], preferred_element_type=jnp.float32)
    m2 = jnp.dot(agg, w_ref[2], preferred_element_type=jnp.float32)
    out = m0 + s1_ref[...] * m1 + s2_ref[...] * m2 + b_ref[...]
    if relu:
        out = jnp.maximum(out, 0.0)
    out_ref[...] = out


def _pna_post(s, sq, mx, mn, deg, s1, s2, W, b, *, relu):
    n, c = s.shape
    out = W.shape[-1]
    w3 = W.reshape(3, 4 * c, out)
    b2 = b.reshape(1, out)
    R = 1000
    grid = (n // R,)
    row = lambda i: (i, 0)
    return pl.pallas_call(
        functools.partial(_post_body, relu=relu),
        grid=grid,
        in_specs=[
            pl.BlockSpec((R, c), row),
            pl.BlockSpec((R, c), row),
            pl.BlockSpec((R, c), row),
            pl.BlockSpec((R, c), row),
            pl.BlockSpec((R, 1), row),
            pl.BlockSpec((R, 1), row),
            pl.BlockSpec((R, 1), row),
            pl.BlockSpec((3, 4 * c, out), lambda i: (0, 0, 0)),
            pl.BlockSpec((1, out), lambda i: (0, 0)),
        ],
        out_specs=pl.BlockSpec((R, out), row),
        out_shape=jax.ShapeDtypeStruct((n, out), jnp.float32),
    )(s, sq, mx, mn, deg, s1, s2, w3, b2)


def kernel(x, edge_index, W1, b1, W2, b2):
    x = x.astype(jnp.float32)
    i32 = jnp.int32
    src = edge_index[0].astype(i32)
    dst = edge_index[1].astype(i32)
    pad = _E_PAD - _E
    srcp = jnp.concatenate([src, jnp.zeros((pad,), i32)])
    dstp = jnp.concatenate([dst, jnp.full((pad,), _NPAD, i32)])

    chunks, counts = _bucketize(srcp, dstp)
    counts = counts.reshape(-1)

    s, q, mx, mn, degr = _moments(x, chunks, counts, 1)
    deg = degr[:64, :_RB].reshape(_NPAD)[:_N]
    log_deg = jnp.log(deg + 1.0)
    delta = jnp.mean(log_deg)
    s1 = (log_deg / delta)[:, None]
    s2 = (delta / jnp.clip(log_deg, 1e-5))[:, None]
    degc = deg[:, None]

    h = _pna_post(_to_full(s, 1), _to_full(q, 1), _to_full(mx, 1),
                  _to_full(mn, 1), degc, s1, s2, W1, b1, relu=True)

    s, q, mx, mn, _ = _moments(h.reshape(_N * 2, 128), chunks, counts, 2)
    return _pna_post(_to_full(s, 2), _to_full(q, 2), _to_full(mx, 2),
                     _to_full(mn, 2), degc, s1, s2, W2, b2, relu=False)


# speedup vs baseline: 2.9674x; 1.0024x over previous
import functools

import jax
import jax.numpy as jnp
from jax import lax
from jax.experimental import pallas as pl
from jax.experimental.pallas import tpu as pltpu
from jax.experimental.pallas import tpu_sc as plsc

_N = 10000
_E = 320000

_NC = 2
_NS = 16
_NW = _NC * _NS
_RB = 160
_NBKT = 65
_NPAD = 64 * _RB
_EPT = _NPAD
_E_PAD = _NW * _EPT
_SB = 1280
_CK = 64
_TSLOT = 161
_MAXC = _NS * _TSLOT


def _bucket_body(src_hbm, dst_hbm, chunks_out, counts_out,
                 st_s, st_d, bs_s, bs_d, fb, cbcnt, fl_smem, nc_smem):
    scid = lax.axis_index("c")
    sid = lax.axis_index("s")
    wid = sid * _NC + scid
    zi = jnp.zeros((16,), jnp.int32)
    iv = lax.iota(jnp.int32, 16)

    for b in range(_NBKT):
        fl_smem[b] = 0
        nc_smem[b] = 0

    def blk(bi, c):
        base = wid * _EPT + bi * _SB
        pltpu.sync_copy(src_hbm.at[pl.ds(base, _SB)], st_s.at[pl.ds(0, _SB)])
        pltpu.sync_copy(dst_hbm.at[pl.ds(base, _SB)], st_d.at[pl.ds(0, _SB)])

        def eb(e, c2):
            s_e = st_s[pl.ds(e, 16)][0]
            d_e = st_d[pl.ds(e, 16)][0]
            b = d_e // _RB
            dl = d_e - b * _RB
            fill = fl_smem[b]
            bs_s[b, pl.ds(fill, 16)] = zi + s_e
            bs_d[b, pl.ds(fill, 16)] = zi + dl

            def flush():
                for j in range(4):
                    sl = pl.ds(j * 16, 16)
                    fb[sl] = bs_s[b, sl]
                    fb[pl.ds(64 + j * 16, 16)] = bs_d[b, sl]
                ncb = nc_smem[b]
                pltpu.sync_copy(fb, chunks_out.at[scid, b, sid * _TSLOT + ncb])
                nc_smem[b] = ncb + 1
                return 0

            nf = lax.cond(fill + 1 >= _CK, flush, lambda: fill + 1)
            fl_smem[b] = nf
            return c2

        lax.fori_loop(0, _SB, eb, 0)
        return c

    lax.fori_loop(0, _EPT // _SB, blk, 0)

    for b in range(_NBKT):
        cbcnt[pl.ds(b, 16)] = zi + (nc_smem[b] * _CK + fl_smem[b])
    pltpu.sync_copy(cbcnt, counts_out.at[scid, sid])

    def fin(b, c):
        fill = fl_smem[b]

        def do():
            for j in range(4):
                sl = pl.ds(j * 16, 16)
                m = (iv + j * 16) < fill
                fb[sl] = jnp.where(m, bs_s[b, sl], 0)
                fb[pl.ds(64 + j * 16, 16)] = jnp.where(m, bs_d[b, sl], _RB)
            ncb = nc_smem[b]
            pltpu.sync_copy(fb, chunks_out.at[scid, b, sid * _TSLOT + ncb])
            nc_smem[b] = ncb + 1
            return 0

        return lax.cond(fill > 0, do, lambda: 0) * 0 + c

    lax.fori_loop(0, _NBKT, fin, 0)


def _bucketize(src, dst):
    mesh = plsc.VectorSubcoreMesh(core_axis_name="c", subcore_axis_name="s",
                                  num_cores=_NC, num_subcores=_NS)
    i32 = jnp.int32
    kern = pl.kernel(
        _bucket_body,
        out_type=(
            jax.ShapeDtypeStruct((_NC, _NBKT, _MAXC, 128), i32),
            jax.ShapeDtypeStruct((_NC, _NS, 128), i32),
        ),
        mesh=mesh,
        scratch_types=[
            pltpu.VMEM((_SB + 16,), i32),
            pltpu.VMEM((_SB + 16,), i32),
            pltpu.VMEM((_NBKT, 80), i32),
            pltpu.VMEM((_NBKT, 80), i32),
            pltpu.VMEM((128,), i32),
            pltpu.VMEM((128,), i32),
            pltpu.SMEM((80,), i32),
            pltpu.SMEM((80,), i32),
        ],
    )
    return kern(src, dst)


def _mom_body(P, xv, chunks, counts, s_out, q_out, mx_out, mn_out, deg_out,
              cntf, wl_a, wl_m, cbufs, gidx, dlb, rows, sa, qa, xa, na, dg,
              semca, semcb, semga, semgb):
    scid = lax.axis_index("c")
    sid = lax.axis_index("s")
    t = sid * _NC + scid
    pltpu.sync_copy(counts, cntf)
    zf = jnp.zeros((16,), jnp.float32)
    zi = jnp.zeros((16,), jnp.int32)
    ninf = jnp.full((16,), -jnp.inf, jnp.float32)
    pinf = jnp.full((16,), jnp.inf, jnp.float32)
    onehot = jnp.where(lax.iota(jnp.int32, 16) == 0, 1.0, 0.0)

    def do_bucket(p, b):
        def ms(r, c):
            for j in range(8):
                sl = pl.ds(j * 16, 16)
                sa[r, sl] = zf
                qa[r, sl] = zf
                xa[r, sl] = ninf
                na[r, sl] = pinf
            return c

        lax.fori_loop(0, _RB + 16, ms, 0)

        def msd(i, c):
            dg[pl.ds(i * 16, 16)] = zf
            return c

        lax.fori_loop(0, 17, msd, 0)

        ntot = 0
        for scsrc in range(_NC):
            def tl_b(tl, n):
                ec = cntf[pl.ds(scsrc * 2048 + tl * 128 + b, 16)][0]
                nch = (ec + _CK - 1) // _CK

                def cb(ch, n2):
                    wl_a[pl.ds(n2, 16)] = zi + (scsrc * 65536 + tl * _TSLOT + ch)
                    wl_m[pl.ds(n2, 16)] = zi + jnp.where(
                        ch == nch - 1, ec - (nch - 1) * _CK, _CK)
                    return n2 + 1

                return lax.fori_loop(0, nch, cb, n)

            ntot = lax.fori_loop(0, _NS, tl_b, ntot)

        def fetch(i, par):
            a = wl_a[pl.ds(i, 16)][0]
            sc = a >> 16
            slot = a & 0xFFFF

            def fa():
                pltpu.async_copy(chunks.at[sc, b, slot], cbufs.at[0], semca)
                return 0

            def fb():
                pltpu.async_copy(chunks.at[sc, b, slot], cbufs.at[1], semcb)
                return 0

            lax.cond(par == 0, fa, fb)

        def wait_cbuf(par):
            def wa():
                pltpu.make_async_copy(chunks.at[0, 0, 0], cbufs.at[0],
                                      semca).wait()
                return 0

            def wb():
                pltpu.make_async_copy(chunks.at[0, 0, 0], cbufs.at[1],
                                      semcb).wait()
                return 0

            lax.cond(par == 0, wa, wb)

        def build_gather(par):
            def ga():
                for j in range(4):
                    sl = pl.ds(j * 16, 16)
                    gidx[0, sl] = cbufs[0, sl] * P + p
                    dlb[0, sl] = cbufs[0, pl.ds(64 + j * 16, 16)]
                pltpu.async_copy(xv.at[gidx.at[0]], rows.at[0], semga)
                return 0

            def gb():
                for j in range(4):
                    sl = pl.ds(j * 16, 16)
                    gidx[1, sl] = cbufs[1, sl] * P + p
                    dlb[1, sl] = cbufs[1, pl.ds(64 + j * 16, 16)]
                pltpu.async_copy(xv.at[gidx.at[1]], rows.at[1], semgb)
                return 0

            lax.cond(par == 0, ga, gb)

        def wait_gather(par):
            def wa():
                pltpu.make_async_copy(xv.at[gidx.at[0]], rows.at[0],
                                      semga).wait()
                return 0

            def wb():
                pltpu.make_async_copy(xv.at[gidx.at[1]], rows.at[1],
                                      semgb).wait()
                return 0

            lax.cond(par == 0, wa, wb)

        def process(i, par):
            m = wl_m[pl.ds(i, 16)][0]
            ng = (m + 15) >> 4

            def gb_(g, c3):
                dlv = dlb[par, pl.ds(g * 16, 16)]
                base = g * 16
                for k in range(16):
                    dl = dlv[k]
                    e = base + k
                    for j in range(8):
                        sl = pl.ds(j * 16, 16)
                        v = rows[par, e, sl]
                        plsc.addupdate(sa.at[dl, sl], v)
                        plsc.addupdate(qa.at[dl, sl], v * v)
                        xa[dl, sl] = jnp.maximum(xa[dl, sl], v)
                        na[dl, sl] = jnp.minimum(na[dl, sl], v)
                    plsc.addupdate(dg.at[pl.ds(dl, 16)], onehot)
                return c3

            lax.fori_loop(0, ng, gb_, 0)

        def have_work():
            fetch(0, 0)

            def it(i, c):
                par = i & 1
                nxt = 1 - par
                wait_cbuf(par)
                build_gather(par)
                lax.cond(i + 1 < ntot,
                         lambda: (fetch(i + 1, nxt), 0)[1], lambda: 0)
                lax.cond(i > 0,
                         lambda: (wait_gather(nxt), process(i - 1, nxt), 0)[2],
                         lambda: 0)
                return c

            lax.fori_loop(0, ntot, it, 0)
            lastpar = (ntot - 1) & 1
            wait_gather(lastpar)
            process(ntot - 1, lastpar)
            return 0

        lax.cond(ntot > 0, have_work, lambda: 0)

        lo = b * _RB
        pltpu.sync_copy(sa.at[pl.ds(0, _RB)], s_out.at[p, pl.ds(lo, _RB)])
        pltpu.sync_copy(qa.at[pl.ds(0, _RB)], q_out.at[p, pl.ds(lo, _RB)])
        pltpu.sync_copy(xa.at[pl.ds(0, _RB)], mx_out.at[p, pl.ds(lo, _RB)])
        pltpu.sync_copy(na.at[pl.ds(0, _RB)], mn_out.at[p, pl.ds(lo, _RB)])

        def wd():
            pltpu.sync_copy(dg.at[pl.ds(0, 256)], deg_out.at[b])
            return 0

        lax.cond(p == 0, wd, lambda: 0)

    def p_body(p, c):
        for k in range(2):
            do_bucket(p, t + 32 * k)
        return c

    lax.fori_loop(0, P, p_body, 0)


def _moments(xv, chunks, counts, P):
    mesh = plsc.VectorSubcoreMesh(core_axis_name="c", subcore_axis_name="s",
                                  num_cores=_NC, num_subcores=_NS)
    f32 = jnp.float32
    i32 = jnp.int32
    kern = pl.kernel(
        functools.partial(_mom_body, P),
        out_type=(
            jax.ShapeDtypeStruct((P, _NPAD, 128), f32),
            jax.ShapeDtypeStruct((P, _NPAD, 128), f32),
            jax.ShapeDtypeStruct((P, _NPAD, 128), f32),
            jax.ShapeDtypeStruct((P, _NPAD, 128), f32),
            jax.ShapeDtypeStruct((_NBKT, 256), f32),
        ),
        mesh=mesh,
        scratch_types=[
            pltpu.VMEM((4096,), i32),
            pltpu.VMEM((5168,), i32),
            pltpu.VMEM((5168,), i32),
            pltpu.VMEM((2, 128), i32),
            pltpu.VMEM((2, 64), i32),
            pltpu.VMEM((2, 80), i32),
            pltpu.VMEM((2, _CK, 128), f32),
            pltpu.VMEM((_RB + 16, 128), f32),
            pltpu.VMEM((_RB + 16, 128), f32),
            pltpu.VMEM((_RB + 16, 128), f32),
            pltpu.VMEM((_RB + 16, 128), f32),
            pltpu.VMEM((272,), f32),
            pltpu.SemaphoreType.DMA,
            pltpu.SemaphoreType.DMA,
            pltpu.SemaphoreType.DMA,
            pltpu.SemaphoreType.DMA,
        ],
    )
    return kern(xv, chunks, counts)


def _to_full(a, P):
    if P == 1:
        return a[0, :_N]
    return jnp.moveaxis(a[:, :_N, :], 0, 1).reshape(_N, P * 128)


def _post_body(sum_ref, sq_ref, mx_ref, mn_ref, deg_ref, s1_ref, s2_ref,
               w_ref, b_ref, out_ref, *, relu):
    deg_c = jnp.maximum(deg_ref[...], 1.0)
    mean = sum_ref[...] / deg_c
    var = jnp.maximum(sq_ref[...] / deg_c - mean * mean, 0.0)
    std = jnp.sqrt(var + 1e-5)
    mx = mx_ref[...]
    mn = mn_ref[...]
    mx = jnp.where(jnp.isfinite(mx), mx, 0.0)
    mn = jnp.where(jnp.isfinite(mn), mn, 0.0)
    agg = jnp.concatenate([mean, mn, mx, std], axis=-1)
    m0 = jnp.dot(agg, w_ref[0], preferred_element_type=jnp.float32)
    m1 = jnp.dot(agg, w_ref[1], preferred_element_type=jnp.float32)
    m2 = jnp.dot(agg, w_ref[2], preferred_element_type=jnp.float32)
    out = m0 + s1_ref[...] * m1 + s2_ref[...] * m2 + b_ref[...]
    if relu:
        out = jnp.maximum(out, 0.0)
    out_ref[...] = out


def _pna_post(s, sq, mx, mn, deg, s1, s2, W, b, *, relu):
    n, c = s.shape
    out = W.shape[-1]
    w3 = W.reshape(3, 4 * c, out)
    b2 = b.reshape(1, out)
    R = 1000
    grid = (n // R,)
    row = lambda i: (i, 0)
    return pl.pallas_call(
        functools.partial(_post_body, relu=relu),
        grid=grid,
        in_specs=[
            pl.BlockSpec((R, c), row),
            pl.BlockSpec((R, c), row),
            pl.BlockSpec((R, c), row),
            pl.BlockSpec((R, c), row),
            pl.BlockSpec((R, 1), row),
            pl.BlockSpec((R, 1), row),
            pl.BlockSpec((R, 1), row),
            pl.BlockSpec((3, 4 * c, out), lambda i: (0, 0, 0)),
            pl.BlockSpec((1, out), lambda i: (0, 0)),
        ],
        out_specs=pl.BlockSpec((R, out), row),
        out_shape=jax.ShapeDtypeStruct((n, out), jnp.float32),
    )(s, sq, mx, mn, deg, s1, s2, w3, b2)


def kernel(x, edge_index, W1, b1, W2, b2):
    x = x.astype(jnp.float32)
    i32 = jnp.int32
    src = edge_index[0].astype(i32)
    dst = edge_index[1].astype(i32)
    pad = _E_PAD - _E
    srcp = jnp.concatenate([src, jnp.zeros((pad,), i32)])
    dstp = jnp.concatenate([dst, jnp.full((pad,), _NPAD, i32)])

    chunks, counts = _bucketize(srcp, dstp)
    counts = counts.reshape(-1)

    s, q, mx, mn, degr = _moments(x, chunks, counts, 1)
    deg = degr[:64, :_RB].reshape(_NPAD)[:_N]
    log_deg = jnp.log(deg + 1.0)
    delta = jnp.mean(log_deg)
    s1 = (log_deg / delta)[:, None]
    s2 = (delta / jnp.clip(log_deg, 1e-5))[:, None]
    degc = deg[:, None]

    h = _pna_post(_to_full(s, 1), _to_full(q, 1), _to_full(mx, 1),
                  _to_full(mn, 1), degc, s1, s2, W1, b1, relu=True)

    s, q, mx, mn, _ = _moments(h.reshape(_N * 2, 128), chunks, counts, 2)
    return _pna_post(_to_full(s, 2), _to_full(q, 2), _to_full(mx, 2),
                     _to_full(mn, 2), degc, s1, s2, W2, b2, relu=False)
